# baseline (speedup 1.0000x reference)
import functools

import jax
import jax.numpy as jnp
from jax.experimental import pallas as pl

_EPS = 0.1
_NEG_INF = float("-inf")
_POS_INF = float("inf")


def _roll_left(x, j):
    return jnp.concatenate([x[:, j:], x[:, :j]], axis=1)


def _roll_right(x, j):
    return jnp.concatenate([x[:, -j:], x[:, :-j]], axis=1)


def _bitonic_argsort_desc(vals, n):
    lane = jax.lax.broadcasted_iota(jnp.int32, (1, n), 1)
    idx = lane
    k = 2
    while k <= n:
        j = k // 2
        while j >= 1:
            lower = (lane // j) % 2 == 0
            dir_desc = (lane // k) % 2 == 0
            pv = jnp.where(lower, _roll_left(vals, j), _roll_right(vals, j))
            pi = jnp.where(lower, _roll_left(idx, j), _roll_right(idx, j))
            want_max = lower == dir_desc
            sel_own = (want_max & (vals >= pv)) | (~want_max & (vals <= pv))
            vals = jnp.where(sel_own, vals, pv)
            idx = jnp.where(sel_own, idx, pi)
            j //= 2
        k *= 2
    return vals, idx


def _reverse_lanes(x, n):
    lane = jax.lax.broadcasted_iota(jnp.int32, (1, n), 1)
    d = 1
    while d < n:
        lower = (lane // d) % 2 == 0
        x = jnp.where(lower, _roll_left(x, d), _roll_right(x, d))
        d *= 2
    return x


def _bitonic_scatter_by_key(keys, payload, n):
    lane = jax.lax.broadcasted_iota(jnp.int32, (1, n), 1)
    k = 2
    while k <= n:
        j = k // 2
        while j >= 1:
            lower = (lane // j) % 2 == 0
            dir_asc = (lane // k) % 2 == 0
            pk = jnp.where(lower, _roll_left(keys, j), _roll_right(keys, j))
            pp = jnp.where(lower, _roll_left(payload, j), _roll_right(payload, j))
            want_min = lower == dir_asc
            sel_own = (want_min & (keys <= pk)) | (~want_min & (keys >= pk))
            keys = jnp.where(sel_own, keys, pk)
            payload = jnp.where(sel_own, payload, pp)
            j //= 2
        k *= 2
    return payload


def _cumsum_row_blocked(y, n, lane):
    c = y
    lane_mod = lane % 128
    for i in range(1, 128):
        prev = jnp.concatenate([c[:, -1:], c[:, :-1]], axis=1)
        c = jnp.where(lane_mod == i, c + prev, c)
    zeros = jnp.zeros_like(c)
    carry = jnp.float32(0.0)
    carry_row = zeros
    for b in range(1, n // 128):
        tot = jnp.sum(jnp.where(lane == b * 128 - 1, c, zeros))
        carry = carry + tot
        carry_row = jnp.where(
            lane >= b * 128, jnp.broadcast_to(carry, c.shape), carry_row
        )
    return c + carry_row


def _isotonic_min_max(means, mask_ge, n):
    m = jnp.where(mask_ge, means, _NEG_INF)
    t = 1
    while t < n:
        shifted = jnp.concatenate(
            [m[:, t:], jnp.full((n, t), _NEG_INF, m.dtype)], axis=1
        )
        m = jnp.maximum(m, shifted)
        t *= 2
    m = jnp.where(mask_ge, m, _POS_INF)
    return jnp.min(m, axis=0, keepdims=True)


def _fused_kernel(
    q_ref,
    doc_ref,
    scores_ref,
    w1_ref,
    b1_ref,
    w2_ref,
    b2_ref,
    w3_ref,
    b3_ref,
    sorted_ref,
    ranks_ref,
    *,
    n,
):
    f32 = jnp.float32
    doc = doc_ref[0]
    q = q_ref[0]
    scores = scores_ref[0]

    sub = jax.lax.broadcasted_iota(jnp.int32, (n, n), 0)
    lane2 = jax.lax.broadcasted_iota(jnp.int32, (n, n), 1)
    ident = (sub == lane2).astype(f32)
    mask_ge = lane2 >= sub
    li = (lane2 - sub + 1).astype(f32)
    len_safe = jnp.where(mask_ge, li, 1.0)
    lane = jax.lax.broadcasted_iota(jnp.int32, (1, n), 1)

    scol = jax.lax.dot_general(
        ident, scores, (((1,), (1,)), ((), ())), preferred_element_type=f32
    )
    feats = jnp.concatenate(
        [jnp.broadcast_to(q, (n, q.shape[1])), doc, scol], axis=1
    )
    h1 = jnp.maximum(
        jax.lax.dot_general(
            feats, w1_ref[...], (((1,), (0,)), ((), ())), preferred_element_type=f32
        )
        + b1_ref[...],
        0.0,
    )
    h2 = jnp.maximum(
        jax.lax.dot_general(
            h1, w2_ref[...], (((1,), (0,)), ((), ())), preferred_element_type=f32
        )
        + b2_ref[...],
        0.0,
    )
    refined = (
        jax.lax.dot_general(
            w3_ref[...], h2, (((1,), (1,)), ((), ())), preferred_element_type=f32
        )
        + b3_ref[...]
    )

    s_desc, perm_desc = _bitonic_argsort_desc(refined, n)

    w = (n - lane).astype(f32) / f32(_EPS)

    c1 = _cumsum_row_blocked(w - s_desc, n, lane)
    cs1 = jnp.concatenate([jnp.zeros((1, 1), f32), c1[:, : n - 1]], axis=1)
    cs1_col = jax.lax.dot_general(
        ident, cs1, (((1,), (1,)), ((), ())), preferred_element_type=f32
    )
    means1 = (c1 - cs1_col) / len_safe
    sol1 = _isotonic_min_max(means1, mask_ge, n)
    sorted_ref[0] = w - sol1

    wr = (n - lane).astype(f32)
    s_rank = _reverse_lanes(s_desc, n) * (-1.0 / _EPS)
    y2 = s_rank - wr
    c2 = _cumsum_row_blocked(y2, n, lane)
    cs2 = jnp.concatenate([jnp.zeros((1, 1), f32), c2[:, : n - 1]], axis=1)
    cs2_col = jax.lax.dot_general(
        ident, cs2, (((1,), (1,)), ((), ())), preferred_element_type=f32
    )
    means2 = (c2 - cs2_col) / len_safe
    dual = _isotonic_min_max(means2, mask_ge, n)
    primal = s_rank - dual
    perm_asc = _reverse_lanes(perm_desc, n)
    ranks_ref[0] = _bitonic_scatter_by_key(perm_asc, primal, n)


def kernel(query_embed, doc_embeds, initial_scores, W1, b1, W2, b2, W3, b3):
    B, N, D = doc_embeds.shape
    H = W2.shape[0]
    f32 = jnp.float32

    q3 = query_embed.reshape(B, 1, D)
    s3 = initial_scores.reshape(B, 1, N)
    b1r = b1.reshape(1, H)
    b2r = b2.reshape(1, H)
    b3r = b3.reshape(1, 1)

    row_spec = lambda shape: pl.BlockSpec(shape, lambda b: (b, 0, 0))
    full_spec = lambda shape: pl.BlockSpec(shape, lambda b: (0, 0))

    sorted3, ranks3 = pl.pallas_call(
        functools.partial(_fused_kernel, n=N),
        grid=(B,),
        in_specs=[
            row_spec((1, 1, D)),
            row_spec((1, N, D)),
            row_spec((1, 1, N)),
            full_spec((2 * D + 1, H)),
            full_spec((1, H)),
            full_spec((H, H)),
            full_spec((1, H)),
            full_spec((1, H)),
            full_spec((1, 1)),
        ],
        out_specs=[
            pl.BlockSpec((1, 1, N), lambda b: (b, 0, 0)),
            pl.BlockSpec((1, 1, N), lambda b: (b, 0, 0)),
        ],
        out_shape=[
            jax.ShapeDtypeStruct((B, 1, N), f32),
            jax.ShapeDtypeStruct((B, 1, N), f32),
        ],
    )(q3, doc_embeds, s3, W1, b1r, W2, b2r, W3.reshape(1, H), b3r)

    return sorted3.reshape(B, N), ranks3.reshape(B, N)


# speedup vs baseline: 3.5688x; 3.5688x over previous
import functools

import jax
import jax.numpy as jnp
from jax.experimental import pallas as pl

_EPS = 0.1
_NEG_INF = float("-inf")
_POS_INF = float("inf")
_ROWS = 4


def _roll_left(x, j):
    return jnp.concatenate([x[:, j:], x[:, :j]], axis=1)


def _roll_right(x, j):
    return jnp.concatenate([x[:, -j:], x[:, :-j]], axis=1)


def _bitonic_argsort_desc(vals, n):
    lane = jax.lax.broadcasted_iota(jnp.int32, (1, n), 1)
    idx = jnp.broadcast_to(lane, vals.shape)
    k = 2
    while k <= n:
        j = k // 2
        while j >= 1:
            lower = (lane // j) % 2 == 0
            dir_desc = (lane // k) % 2 == 0
            pv = jnp.where(lower, _roll_left(vals, j), _roll_right(vals, j))
            pi = jnp.where(lower, _roll_left(idx, j), _roll_right(idx, j))
            want_max = lower == dir_desc
            sel_own = (want_max & (vals >= pv)) | (~want_max & (vals <= pv))
            vals = jnp.where(sel_own, vals, pv)
            idx = jnp.where(sel_own, idx, pi)
            j //= 2
        k *= 2
    return vals, idx


def _reverse_lanes(x, n):
    lane = jax.lax.broadcasted_iota(jnp.int32, (1, n), 1)
    d = 1
    while d < n:
        lower = (lane // d) % 2 == 0
        x = jnp.where(lower, _roll_left(x, d), _roll_right(x, d))
        d *= 2
    return x


def _bitonic_scatter_by_key(keys, payload, n):
    lane = jax.lax.broadcasted_iota(jnp.int32, (1, n), 1)
    k = 2
    while k <= n:
        j = k // 2
        while j >= 1:
            lower = (lane // j) % 2 == 0
            dir_asc = (lane // k) % 2 == 0
            pk = jnp.where(lower, _roll_left(keys, j), _roll_right(keys, j))
            pp = jnp.where(lower, _roll_left(payload, j), _roll_right(payload, j))
            want_min = lower == dir_asc
            sel_own = (want_min & (keys <= pk)) | (~want_min & (keys >= pk))
            keys = jnp.where(sel_own, keys, pk)
            payload = jnp.where(sel_own, payload, pp)
            j //= 2
        k *= 2
    return payload


def _cumsum_rows_blocked(y, n):
    lane = jax.lax.broadcasted_iota(jnp.int32, (1, n), 1)
    lane_mod = lane % 128
    c = y
    for i in range(1, 128):
        prev = jnp.concatenate([c[:, -1:], c[:, :-1]], axis=1)
        c = jnp.where(lane_mod == i, c + prev, c)
    zeros = jnp.zeros_like(c)
    carry_col = jnp.zeros((y.shape[0], 1), y.dtype)
    carry_row = zeros
    for b in range(1, n // 128):
        tot = jnp.sum(
            jnp.where(lane == b * 128 - 1, c, zeros), axis=1, keepdims=True
        )
        carry_col = carry_col + tot
        carry_row = jnp.where(lane >= b * 128, carry_col, carry_row)
    return c + carry_row


def _isotonic_stacked(c, cs, ident, mask_m, len_safe_m, n, rows):
    f32 = jnp.float32
    cs_col = jnp.concatenate(
        [
            jax.lax.dot_general(
                ident, cs[i : i + 1], (((1,), (1,)), ((), ())),
                preferred_element_type=f32,
            )
            for i in range(rows)
        ],
        axis=0,
    )
    cb = jnp.concatenate(
        [jnp.broadcast_to(c[i : i + 1], (n, n)) for i in range(rows)], axis=0
    )
    means = (cb - cs_col) / len_safe_m
    m = jnp.where(mask_m, means, _NEG_INF)
    t = 1
    while t < n:
        shifted = jnp.concatenate(
            [m[:, t:], jnp.full((rows * n, t), _NEG_INF, m.dtype)], axis=1
        )
        m = jnp.maximum(m, shifted)
        t *= 2
    m = jnp.where(mask_m, m, _POS_INF)
    return jnp.concatenate(
        [jnp.min(m[i * n : (i + 1) * n], axis=0, keepdims=True) for i in range(rows)],
        axis=0,
    )


def _fused_kernel(
    q_ref,
    doc_ref,
    scores_ref,
    w1_ref,
    b1_ref,
    w2_ref,
    b2_ref,
    w3_ref,
    b3_ref,
    sorted_ref,
    ranks_ref,
    *,
    n,
    rows,
):
    f32 = jnp.float32
    q = q_ref[0]
    scores = scores_ref[0]
    d_feat = q.shape[1]

    sub = jax.lax.broadcasted_iota(jnp.int32, (n, n), 0)
    lane2 = jax.lax.broadcasted_iota(jnp.int32, (n, n), 1)
    ident = (sub == lane2).astype(f32)
    sub_m = jax.lax.broadcasted_iota(jnp.int32, (rows * n, n), 0) % n
    lane_m = jax.lax.broadcasted_iota(jnp.int32, (rows * n, n), 1)
    mask_m = lane_m >= sub_m
    len_safe_m = jnp.where(mask_m, (lane_m - sub_m + 1).astype(f32), 1.0)
    lane = jax.lax.broadcasted_iota(jnp.int32, (1, n), 1)

    scol = jnp.concatenate(
        [
            jax.lax.dot_general(
                ident, scores[i : i + 1], (((1,), (1,)), ((), ())),
                preferred_element_type=f32,
            )
            for i in range(rows)
        ],
        axis=0,
    )
    qb = jnp.concatenate(
        [jnp.broadcast_to(q[i : i + 1], (n, d_feat)) for i in range(rows)], axis=0
    )
    doc = jnp.concatenate([doc_ref[i] for i in range(rows)], axis=0)
    feats = jnp.concatenate([qb, doc, scol], axis=1)
    h1 = jnp.maximum(
        jax.lax.dot_general(
            feats, w1_ref[...], (((1,), (0,)), ((), ())), preferred_element_type=f32
        )
        + b1_ref[...],
        0.0,
    )
    h2 = jnp.maximum(
        jax.lax.dot_general(
            h1, w2_ref[...], (((1,), (0,)), ((), ())), preferred_element_type=f32
        )
        + b2_ref[...],
        0.0,
    )
    refined = (
        jnp.concatenate(
            [
                jax.lax.dot_general(
                    w3_ref[...], h2[i * n : (i + 1) * n], (((1,), (1,)), ((), ())),
                    preferred_element_type=f32,
                )
                for i in range(rows)
            ],
            axis=0,
        )
        + b3_ref[...]
    )

    s_desc, perm_desc = _bitonic_argsort_desc(refined, n)

    w = (n - lane).astype(f32) / f32(_EPS)
    wr = (n - lane).astype(f32)
    y1 = w - s_desc
    s_rank = _reverse_lanes(s_desc, n) * (-1.0 / _EPS)
    y2 = s_rank - wr

    cst = _cumsum_rows_blocked(jnp.concatenate([y1, y2], axis=0), n)
    c1 = cst[:rows]
    c2 = cst[rows:]
    zcol = jnp.zeros((rows, 1), f32)
    cs1 = jnp.concatenate([zcol, c1[:, : n - 1]], axis=1)
    cs2 = jnp.concatenate([zcol, c2[:, : n - 1]], axis=1)

    viol = jnp.any(y1[:, 1:] > y1[:, : n - 1] - 2.0)

    @pl.when(jnp.logical_not(viol))
    def _cheap_sort():
        sorted_ref[0] = w - (c1 - cs1)

    @pl.when(viol)
    def _full_sort():
        sol1 = _isotonic_stacked(c1, cs1, ident, mask_m, len_safe_m, n, rows)
        sorted_ref[0] = w - sol1

    dual = _isotonic_stacked(c2, cs2, ident, mask_m, len_safe_m, n, rows)
    primal = s_rank - dual
    perm_asc = _reverse_lanes(perm_desc, n)
    ranks_ref[0] = _bitonic_scatter_by_key(perm_asc, primal, n)


def kernel(query_embed, doc_embeds, initial_scores, W1, b1, W2, b2, W3, b3):
    B, N, D = doc_embeds.shape
    H = W2.shape[0]
    f32 = jnp.float32
    R = _ROWS
    G = B // R

    q3 = query_embed.reshape(G, R, D)
    s3 = initial_scores.reshape(G, R, N)
    b1r = b1.reshape(1, H)
    b2r = b2.reshape(1, H)
    b3r = b3.reshape(1, 1)

    full_spec = lambda shape: pl.BlockSpec(shape, lambda b: (0, 0))

    sorted3, ranks3 = pl.pallas_call(
        functools.partial(_fused_kernel, n=N, rows=R),
        grid=(G,),
        in_specs=[
            pl.BlockSpec((1, R, D), lambda b: (b, 0, 0)),
            pl.BlockSpec((R, N, D), lambda b: (b, 0, 0)),
            pl.BlockSpec((1, R, N), lambda b: (b, 0, 0)),
            full_spec((2 * D + 1, H)),
            full_spec((1, H)),
            full_spec((H, H)),
            full_spec((1, H)),
            full_spec((1, H)),
            full_spec((1, 1)),
        ],
        out_specs=[
            pl.BlockSpec((1, R, N), lambda b: (b, 0, 0)),
            pl.BlockSpec((1, R, N), lambda b: (b, 0, 0)),
        ],
        out_shape=[
            jax.ShapeDtypeStruct((G, R, N), f32),
            jax.ShapeDtypeStruct((G, R, N), f32),
        ],
    )(q3, doc_embeds, s3, W1, b1r, W2, b2r, W3.reshape(1, H), b3r)

    return sorted3.reshape(B, N), ranks3.reshape(B, N)


# speedup vs baseline: 5.4761x; 1.5344x over previous
import functools

import jax
import jax.numpy as jnp
from jax.experimental import pallas as pl

_EPS = 0.1
_NEG_INF = float("-inf")
_POS_INF = float("inf")
_ROWS = 8


def _roll_left(x, j):
    return jnp.concatenate([x[:, j:], x[:, :j]], axis=1)


def _roll_right(x, j):
    return jnp.concatenate([x[:, -j:], x[:, :-j]], axis=1)


def _bitonic_argsort_desc(vals, n):
    lane = jax.lax.broadcasted_iota(jnp.int32, (1, n), 1)
    idx = jnp.broadcast_to(lane, vals.shape)
    k = 2
    while k <= n:
        j = k // 2
        while j >= 1:
            lower = (lane // j) % 2 == 0
            dir_desc = (lane // k) % 2 == 0
            pv = jnp.where(lower, _roll_left(vals, j), _roll_right(vals, j))
            pi = jnp.where(lower, _roll_left(idx, j), _roll_right(idx, j))
            want_max = lower == dir_desc
            sel_own = (want_max & (vals >= pv)) | (~want_max & (vals <= pv))
            vals = jnp.where(sel_own, vals, pv)
            idx = jnp.where(sel_own, idx, pi)
            j //= 2
        k *= 2
    return vals, idx


def _reverse_lanes(x, n):
    lane = jax.lax.broadcasted_iota(jnp.int32, (1, n), 1)
    d = 1
    while d < n:
        lower = (lane // d) % 2 == 0
        x = jnp.where(lower, _roll_left(x, d), _roll_right(x, d))
        d *= 2
    return x


def _bitonic_scatter_by_key(keys, payload, n):
    lane = jax.lax.broadcasted_iota(jnp.int32, (1, n), 1)
    k = 2
    while k <= n:
        j = k // 2
        while j >= 1:
            lower = (lane // j) % 2 == 0
            dir_asc = (lane // k) % 2 == 0
            pk = jnp.where(lower, _roll_left(keys, j), _roll_right(keys, j))
            pp = jnp.where(lower, _roll_left(payload, j), _roll_right(payload, j))
            want_min = lower == dir_asc
            sel_own = (want_min & (keys <= pk)) | (~want_min & (keys >= pk))
            keys = jnp.where(sel_own, keys, pk)
            payload = jnp.where(sel_own, payload, pp)
            j //= 2
        k *= 2
    return payload


def _cumsum_rows_blocked(y, n):
    lane = jax.lax.broadcasted_iota(jnp.int32, (1, n), 1)
    lane_mod = lane % 128
    c = y
    for i in range(1, 128):
        prev = jnp.concatenate([c[:, -1:], c[:, :-1]], axis=1)
        c = jnp.where(lane_mod == i, c + prev, c)
    zeros = jnp.zeros_like(c)
    carry_col = jnp.zeros((y.shape[0], 1), y.dtype)
    carry_row = zeros
    for b in range(1, n // 128):
        tot = jnp.sum(
            jnp.where(lane == b * 128 - 1, c, zeros), axis=1, keepdims=True
        )
        carry_col = carry_col + tot
        carry_row = jnp.where(lane >= b * 128, carry_col, carry_row)
    return c + carry_row


def _isotonic_stacked(c, cs, ident, mask_m, len_safe_m, n, rows):
    f32 = jnp.float32
    cs_col = jnp.concatenate(
        [
            jax.lax.dot_general(
                ident, cs[i : i + 1], (((1,), (1,)), ((), ())),
                preferred_element_type=f32,
            )
            for i in range(rows)
        ],
        axis=0,
    )
    cb = jnp.concatenate(
        [jnp.broadcast_to(c[i : i + 1], (n, n)) for i in range(rows)], axis=0
    )
    means = (cb - cs_col) / len_safe_m
    m = jnp.where(mask_m, means, _NEG_INF)
    t = 1
    while t < n:
        shifted = jnp.concatenate(
            [m[:, t:], jnp.full((rows * n, t), _NEG_INF, m.dtype)], axis=1
        )
        m = jnp.maximum(m, shifted)
        t *= 2
    m = jnp.where(mask_m, m, _POS_INF)
    return jnp.concatenate(
        [jnp.min(m[i * n : (i + 1) * n], axis=0, keepdims=True) for i in range(rows)],
        axis=0,
    )


def _fused_kernel(
    q_ref,
    doc_ref,
    scores_ref,
    w1_ref,
    b1_ref,
    w2_ref,
    b2_ref,
    w3_ref,
    b3_ref,
    sorted_ref,
    ranks_ref,
    *,
    n,
    rows,
):
    f32 = jnp.float32
    q = q_ref[0]
    scores = scores_ref[0]
    d_feat = q.shape[1]

    sub = jax.lax.broadcasted_iota(jnp.int32, (n, n), 0)
    lane2 = jax.lax.broadcasted_iota(jnp.int32, (n, n), 1)
    ident = (sub == lane2).astype(f32)
    sub_m = jax.lax.broadcasted_iota(jnp.int32, (rows * n, n), 0) % n
    lane_m = jax.lax.broadcasted_iota(jnp.int32, (rows * n, n), 1)
    mask_m = lane_m >= sub_m
    len_safe_m = jnp.where(mask_m, (lane_m - sub_m + 1).astype(f32), 1.0)
    lane = jax.lax.broadcasted_iota(jnp.int32, (1, n), 1)

    scol = jnp.concatenate(
        [
            jax.lax.dot_general(
                ident, scores[i : i + 1], (((1,), (1,)), ((), ())),
                preferred_element_type=f32,
            )
            for i in range(rows)
        ],
        axis=0,
    )
    qb = jnp.concatenate(
        [jnp.broadcast_to(q[i : i + 1], (n, d_feat)) for i in range(rows)], axis=0
    )
    doc = jnp.concatenate([doc_ref[i] for i in range(rows)], axis=0)
    feats = jnp.concatenate([qb, doc, scol], axis=1)
    h1 = jnp.maximum(
        jax.lax.dot_general(
            feats, w1_ref[...], (((1,), (0,)), ((), ())), preferred_element_type=f32
        )
        + b1_ref[...],
        0.0,
    )
    h2 = jnp.maximum(
        jax.lax.dot_general(
            h1, w2_ref[...], (((1,), (0,)), ((), ())), preferred_element_type=f32
        )
        + b2_ref[...],
        0.0,
    )
    refined = (
        jnp.concatenate(
            [
                jax.lax.dot_general(
                    w3_ref[...], h2[i * n : (i + 1) * n], (((1,), (1,)), ((), ())),
                    preferred_element_type=f32,
                )
                for i in range(rows)
            ],
            axis=0,
        )
        + b3_ref[...]
    )

    s_desc, perm_desc = _bitonic_argsort_desc(refined, n)

    w = (n - lane).astype(f32) / f32(_EPS)
    wr = (n - lane).astype(f32)
    y1 = w - s_desc
    s_rank = _reverse_lanes(s_desc, n) * (-1.0 / _EPS)
    y2 = s_rank - wr

    cst = _cumsum_rows_blocked(jnp.concatenate([y1, y2], axis=0), n)
    c1 = cst[:rows]
    c2 = cst[rows:]
    zcol = jnp.zeros((rows, 1), f32)
    cs1 = jnp.concatenate([zcol, c1[:, : n - 1]], axis=1)
    cs2 = jnp.concatenate([zcol, c2[:, : n - 1]], axis=1)

    viol = jnp.any(y1[:, 1:] > y1[:, : n - 1] - 2.0)

    @pl.when(jnp.logical_not(viol))
    def _cheap_sort():
        sorted_ref[0] = w - (c1 - cs1)

    @pl.when(viol)
    def _full_sort():
        sol1 = _isotonic_stacked(c1, cs1, ident, mask_m, len_safe_m, n, rows)
        sorted_ref[0] = w - sol1

    dual = _isotonic_stacked(c2, cs2, ident, mask_m, len_safe_m, n, rows)
    primal = s_rank - dual
    perm_asc = _reverse_lanes(perm_desc, n)
    ranks_ref[0] = _bitonic_scatter_by_key(perm_asc, primal, n)


def kernel(query_embed, doc_embeds, initial_scores, W1, b1, W2, b2, W3, b3):
    B, N, D = doc_embeds.shape
    H = W2.shape[0]
    f32 = jnp.float32
    R = _ROWS
    G = B // R

    q3 = query_embed.reshape(G, R, D)
    s3 = initial_scores.reshape(G, R, N)
    b1r = b1.reshape(1, H)
    b2r = b2.reshape(1, H)
    b3r = b3.reshape(1, 1)

    full_spec = lambda shape: pl.BlockSpec(shape, lambda b: (0, 0))

    sorted3, ranks3 = pl.pallas_call(
        functools.partial(_fused_kernel, n=N, rows=R),
        grid=(G,),
        in_specs=[
            pl.BlockSpec((1, R, D), lambda b: (b, 0, 0)),
            pl.BlockSpec((R, N, D), lambda b: (b, 0, 0)),
            pl.BlockSpec((1, R, N), lambda b: (b, 0, 0)),
            full_spec((2 * D + 1, H)),
            full_spec((1, H)),
            full_spec((H, H)),
            full_spec((1, H)),
            full_spec((1, H)),
            full_spec((1, 1)),
        ],
        out_specs=[
            pl.BlockSpec((1, R, N), lambda b: (b, 0, 0)),
            pl.BlockSpec((1, R, N), lambda b: (b, 0, 0)),
        ],
        out_shape=[
            jax.ShapeDtypeStruct((G, R, N), f32),
            jax.ShapeDtypeStruct((G, R, N), f32),
        ],
    )(q3, doc_embeds, s3, W1, b1r, W2, b2r, W3.reshape(1, H), b3r)

    return sorted3.reshape(B, N), ranks3.reshape(B, N)
